# baseline (speedup 1.0000x reference)
import functools

import jax
import jax.numpy as jnp
from jax import lax
from jax.experimental import pallas as pl
from jax.experimental.pallas import tpu as pltpu
from jax.experimental.pallas import tpu_sc as plsc

E = 8
K = 2
D = 1024
F = 4096
T = 2048
CAP = 320
NSLOT = E * CAP
TRASH = NSLOT
LANES = 16

NC = 2
NS = 16
NW = NC * NS
TC_PER_TILE = (K * T) // NS
ROWS_PER_W = NSLOT // NW
TOK_PER_W = T // NW

FB = 1024
NF = F // FB


def _routing_body(x_ref, wr_ref, slot_ref, d12_ref, w12_ref):
    x = x_ref[...]
    wr = wr_ref[...]
    logits = jnp.dot(x, wr, preferred_element_type=jnp.float32)
    m = jnp.max(logits, axis=1, keepdims=True)
    ex = jnp.exp(logits - m)
    probs = ex / jnp.sum(ex, axis=1, keepdims=True)

    idx8 = lax.broadcasted_iota(jnp.int32, (T, E), 1)
    m1 = jnp.max(probs, axis=1, keepdims=True)
    e1 = jnp.min(jnp.where(probs == m1, idx8, E), axis=1)
    p1 = m1[:, 0]
    mask0 = (idx8 == e1[:, None]).astype(jnp.float32)
    probs2 = jnp.where(mask0 > 0.0, -1.0, probs)
    m2 = jnp.max(probs2, axis=1, keepdims=True)
    e2 = jnp.min(jnp.where(probs2 == m2, idx8, E), axis=1)
    p2 = m2[:, 0]
    mask1 = (idx8 == e2[:, None]).astype(jnp.float32)

    tri = (lax.broadcasted_iota(jnp.int32, (T, T), 1)
           <= lax.broadcasted_iota(jnp.int32, (T, T), 0)).astype(jnp.bfloat16)
    both = jnp.concatenate([mask0, mask1], axis=1).astype(jnp.bfloat16)
    csum = jnp.dot(tri, both, preferred_element_type=jnp.float32)
    c0 = csum[:, :E]
    c1k = csum[:, E:]
    pos0 = jnp.sum(c0 * mask0, axis=1) - 1.0
    tot0 = jnp.sum(mask0, axis=0)
    c1 = c1k + tot0[None, :]
    pos1 = jnp.sum(c1 * mask1, axis=1) - 1.0

    pos0i = pos0.astype(jnp.int32)
    pos1i = pos1.astype(jnp.int32)
    valid0 = pos0i < CAP
    valid1 = pos1i < CAP
    slot0 = jnp.where(valid0, e1 * CAP + pos0i, TRASH)
    slot1 = jnp.where(valid1, e2 * CAP + pos1i, TRASH)

    slot_ref[0, :] = slot0
    slot_ref[1, :] = slot1
    tok = lax.broadcasted_iota(jnp.int32, (T,), 0)
    d12_ref[...] = jnp.stack(
        [jnp.where(valid0, slot0, tok), jnp.where(valid1, slot1, tok)],
        axis=1)
    ones = jnp.ones((1, LANES), jnp.float32)
    w1b = jnp.where(valid0, p1, 0.0)[:, None] * ones
    w2b = jnp.where(valid1, p2, 0.0)[:, None] * ones
    w12_ref[...] = jnp.stack([w1b, w2b], axis=1)


def _routing(x2d, w_router):
    return pl.pallas_call(
        _routing_body,
        out_shape=(
            jax.ShapeDtypeStruct((K, T), jnp.int32),
            jax.ShapeDtypeStruct((T, K), jnp.int32),
            jax.ShapeDtypeStruct((T, K, LANES), jnp.float32),
        ),
    )(x2d, w_router)


_SC_MESH = plsc.VectorSubcoreMesh(core_axis_name="c", subcore_axis_name="s")


@functools.partial(
    pl.kernel,
    out_type=jax.ShapeDtypeStruct((NSLOT, D), jnp.float32),
    mesh=_SC_MESH,
    scratch_types=[
        pltpu.VMEM((128,), jnp.int32),
        pltpu.VMEM((128,), jnp.int32),
        pltpu.VMEM((128,), jnp.int32),
        pltpu.VMEM((ROWS_PER_W,), jnp.int32),
        pltpu.VMEM((ROWS_PER_W, D), jnp.float32),
        pltpu.VMEM_SHARED((NSLOT + 1,), jnp.int32),
        pltpu.SemaphoreType.DMA,
    ],
)
def _dispatch(slot_hbm, x_hbm, out_hbm, sa, sb, tid, gidx, rows, st_sh, sem):
    cid = lax.axis_index("c")
    sid = lax.axis_index("s")
    base = sid * TC_PER_TILE
    pltpu.sync_copy(slot_hbm.at[pl.ds(base, 128)], sa)
    pltpu.sync_copy(slot_hbm.at[pl.ds(base + 128, 128)], sb)
    for half, sref in ((0, sa), (1, sb)):
        for mvec in range(8):
            off = half * 128 + mvec * LANES
            tid[pl.ds(mvec * LANES, LANES)] = (
                (base + off + lax.iota(jnp.int32, LANES)) & (T - 1))
        pltpu.sync_copy(tid, st_sh.at[sref])
    plsc.subcore_barrier()
    wid = sid * NC + cid
    gbase = wid * ROWS_PER_W
    pltpu.sync_copy(st_sh.at[pl.ds(gbase, ROWS_PER_W)], gidx)
    for mvec in range(ROWS_PER_W // LANES):
        sl = pl.ds(mvec * LANES, LANES)
        gidx[sl] = jnp.minimum(jnp.maximum(gidx[sl], 0), T - 1)
    pltpu.async_copy(x_hbm.at[gidx], rows, sem).wait()
    pltpu.sync_copy(rows, out_hbm.at[pl.ds(gbase, ROWS_PER_W)])


def _ffn_body(xe_ref, win_ref, wout_ref, y_ref):
    f = pl.program_id(1)
    x16 = xe_ref[...].astype(jnp.bfloat16)
    h = jnp.maximum(
        jnp.dot(x16, win_ref[0].astype(jnp.bfloat16),
                preferred_element_type=jnp.float32),
        0.0)
    part = jnp.dot(h.astype(jnp.bfloat16), wout_ref[0].astype(jnp.bfloat16),
                   preferred_element_type=jnp.float32)

    @pl.when(f == 0)
    def _():
        y_ref[...] = part

    @pl.when(f > 0)
    def _():
        y_ref[...] = y_ref[...] + part


def _ffn(ei, w_in, w_out):
    return pl.pallas_call(
        _ffn_body,
        grid=(E, NF),
        in_specs=[
            pl.BlockSpec((CAP, D), lambda e, f: (e, 0)),
            pl.BlockSpec((1, D, FB), lambda e, f: (e, 0, f)),
            pl.BlockSpec((1, FB, D), lambda e, f: (e, f, 0)),
        ],
        out_specs=pl.BlockSpec((CAP, D), lambda e, f: (e, 0)),
        out_shape=jax.ShapeDtypeStruct((NSLOT, D), jnp.float32),
        compiler_params=pltpu.CompilerParams(
            dimension_semantics=("arbitrary", "arbitrary")),
    )(ei, w_in, w_out)


_CB = 16
_NB = TOK_PER_W // _CB


@functools.partial(
    pl.kernel,
    out_type=jax.ShapeDtypeStruct((T, D), jnp.float32),
    mesh=_SC_MESH,
    scratch_types=[
        pltpu.VMEM((2, 2 * _CB), jnp.int32),
        pltpu.VMEM((2, 2 * _CB, LANES), jnp.float32),
        pltpu.VMEM((2 * _CB, D), jnp.float32),
        pltpu.VMEM((2 * _CB, D), jnp.float32),
        pltpu.VMEM((_CB, D), jnp.float32),
        pltpu.VMEM((_CB, D), jnp.float32),
        pltpu.SemaphoreType.DMA,
        pltpu.SemaphoreType.DMA,
        pltpu.SemaphoreType.DMA,
        pltpu.SemaphoreType.DMA,
    ],
)
def _combine(y_hbm, d12_hbm, w12_hbm, out_hbm,
             didx, wv, rp0, rp1, ob0, ob1, g0, g1, o0, o1):
    cid = lax.axis_index("c")
    sid = lax.axis_index("s")
    wid = sid * NC + cid
    base = wid * TOK_PER_W
    rps = (rp0, rp1)
    obs = (ob0, ob1)
    gsems = (g0, g1)
    osems = (o0, o1)

    def stage(b):
        par = b % 2
        tb = base + b * _CB
        pltpu.sync_copy(d12_hbm.at[pl.ds(2 * tb, 2 * _CB)], didx.at[par])
        pltpu.sync_copy(w12_hbm.at[pl.ds(2 * tb, 2 * _CB)], wv.at[par])
        return pltpu.async_copy(y_hbm.at[didx.at[par]], rps[par], gsems[par])

    cp = stage(0)
    pend = [None, None]
    for b in range(_NB):
        par = b % 2
        cp.wait()
        if b + 1 < _NB:
            cp = stage(b + 1)
        if pend[par] is not None:
            pend[par].wait()
        rp, ob = rps[par], obs[par]
        for i in range(_CB):
            w1row = wv[par, 2 * i, :]
            w2row = wv[par, 2 * i + 1, :]

            def vbody(v, carry, i=i, w1row=w1row, w2row=w2row, rp=rp, ob=ob):
                off = v * (8 * LANES)
                for u in range(8):
                    sl = pl.ds(off + u * LANES, LANES)
                    ob[i, sl] = rp[2 * i, sl] * w1row + rp[2 * i + 1, sl] * w2row
                return carry

            lax.fori_loop(0, D // (8 * LANES), vbody, 0)
        pend[par] = pltpu.async_copy(
            ob, out_hbm.at[pl.ds(base + b * _CB, _CB)], osems[par])
    for h in pend:
        if h is not None:
            h.wait()


def kernel(input_emb, w_router, w_in, w_out):
    x2d = input_emb.reshape(T, D).astype(jnp.float32)
    slot_km, d12, w12 = _routing(x2d, w_router)
    slot_flat = slot_km.reshape(K * T)
    ei = _dispatch(slot_flat, x2d)
    y = _ffn(ei, w_in, w_out)
    out = _combine(y, d12.reshape(K * T), w12.reshape(K * T, LANES))
    return out.reshape(1, T, D)


# speedup vs baseline: 1.0207x; 1.0207x over previous
import functools

import jax
import jax.numpy as jnp
from jax import lax
from jax.experimental import pallas as pl
from jax.experimental.pallas import tpu as pltpu
from jax.experimental.pallas import tpu_sc as plsc

E = 8
K = 2
D = 1024
F = 4096
T = 2048
CAP = 320
NSLOT = E * CAP
TRASH = NSLOT
LANES = 16

NC = 2
NS = 16
NW = NC * NS
TC_PER_TILE = (K * T) // NS
ROWS_PER_W = NSLOT // NW
TOK_PER_W = T // NW

FB = 1024
NF = F // FB


def _routing_body(x_ref, wr_ref, slot_ref, d12_ref, w12_ref):
    x = x_ref[...]
    wr = wr_ref[...]
    logits = jnp.dot(x, wr, preferred_element_type=jnp.float32)
    m = jnp.max(logits, axis=1, keepdims=True)
    ex = jnp.exp(logits - m)
    probs = ex / jnp.sum(ex, axis=1, keepdims=True)

    idx8 = lax.broadcasted_iota(jnp.int32, (T, E), 1)
    m1 = jnp.max(probs, axis=1, keepdims=True)
    e1 = jnp.min(jnp.where(probs == m1, idx8, E), axis=1)
    p1 = m1[:, 0]
    mask0 = (idx8 == e1[:, None]).astype(jnp.float32)
    probs2 = jnp.where(mask0 > 0.0, -1.0, probs)
    m2 = jnp.max(probs2, axis=1, keepdims=True)
    e2 = jnp.min(jnp.where(probs2 == m2, idx8, E), axis=1)
    p2 = m2[:, 0]
    mask1 = (idx8 == e2[:, None]).astype(jnp.float32)

    tri = (lax.broadcasted_iota(jnp.int32, (T, T), 1)
           <= lax.broadcasted_iota(jnp.int32, (T, T), 0)).astype(jnp.float32)
    both = jnp.concatenate([mask0, mask1], axis=1)
    csum = jnp.dot(tri, both, preferred_element_type=jnp.float32)
    c0 = csum[:, :E]
    c1k = csum[:, E:]
    pos0 = jnp.sum(c0 * mask0, axis=1) - 1.0
    tot0 = jnp.sum(mask0, axis=0)
    c1 = c1k + tot0[None, :]
    pos1 = jnp.sum(c1 * mask1, axis=1) - 1.0

    pos0i = pos0.astype(jnp.int32)
    pos1i = pos1.astype(jnp.int32)
    valid0 = pos0i < CAP
    valid1 = pos1i < CAP
    slot0 = jnp.where(valid0, e1 * CAP + pos0i, TRASH)
    slot1 = jnp.where(valid1, e2 * CAP + pos1i, TRASH)

    slot_ref[0, :] = slot0
    slot_ref[1, :] = slot1
    tok = lax.broadcasted_iota(jnp.int32, (T,), 0)
    d12_ref[...] = jnp.stack(
        [jnp.where(valid0, slot0, tok), jnp.where(valid1, slot1, tok)],
        axis=1)
    ones = jnp.ones((1, LANES), jnp.float32)
    w1b = jnp.where(valid0, p1, 0.0)[:, None] * ones
    w2b = jnp.where(valid1, p2, 0.0)[:, None] * ones
    w12_ref[...] = jnp.stack([w1b, w2b], axis=1)


def _routing(x2d, w_router):
    return pl.pallas_call(
        _routing_body,
        out_shape=(
            jax.ShapeDtypeStruct((K, T), jnp.int32),
            jax.ShapeDtypeStruct((T, K), jnp.int32),
            jax.ShapeDtypeStruct((T, K, LANES), jnp.float32),
        ),
    )(x2d, w_router)


_SC_MESH = plsc.VectorSubcoreMesh(core_axis_name="c", subcore_axis_name="s")


@functools.partial(
    pl.kernel,
    out_type=jax.ShapeDtypeStruct((NSLOT, D), jnp.float32),
    mesh=_SC_MESH,
    scratch_types=[
        pltpu.VMEM((128,), jnp.int32),
        pltpu.VMEM((128,), jnp.int32),
        pltpu.VMEM((128,), jnp.int32),
        pltpu.VMEM((ROWS_PER_W,), jnp.int32),
        pltpu.VMEM((ROWS_PER_W, D), jnp.float32),
        pltpu.VMEM_SHARED((NSLOT + 1,), jnp.int32),
        pltpu.SemaphoreType.DMA,
    ],
)
def _dispatch(slot_hbm, x_hbm, out_hbm, sa, sb, tid, gidx, rows, st_sh, sem):
    cid = lax.axis_index("c")
    sid = lax.axis_index("s")
    base = sid * TC_PER_TILE
    pltpu.sync_copy(slot_hbm.at[pl.ds(base, 128)], sa)
    pltpu.sync_copy(slot_hbm.at[pl.ds(base + 128, 128)], sb)
    for half, sref in ((0, sa), (1, sb)):
        for mvec in range(8):
            off = half * 128 + mvec * LANES
            tid[pl.ds(mvec * LANES, LANES)] = (
                (base + off + lax.iota(jnp.int32, LANES)) & (T - 1))
        pltpu.sync_copy(tid, st_sh.at[sref])
    plsc.subcore_barrier()
    wid = sid * NC + cid
    gbase = wid * ROWS_PER_W
    pltpu.sync_copy(st_sh.at[pl.ds(gbase, ROWS_PER_W)], gidx)
    for mvec in range(ROWS_PER_W // LANES):
        sl = pl.ds(mvec * LANES, LANES)
        gidx[sl] = jnp.minimum(jnp.maximum(gidx[sl], 0), T - 1)
    pltpu.async_copy(x_hbm.at[gidx], rows, sem).wait()
    pltpu.sync_copy(rows, out_hbm.at[pl.ds(gbase, ROWS_PER_W)])


def _ffn_body(xe_ref, win_ref, wout_ref, y_ref):
    f = pl.program_id(1)
    h = jnp.maximum(
        jnp.dot(xe_ref[...], win_ref[0], preferred_element_type=jnp.float32),
        0.0)
    part = jnp.dot(h, wout_ref[0], preferred_element_type=jnp.float32)

    @pl.when(f == 0)
    def _():
        y_ref[...] = part

    @pl.when(f > 0)
    def _():
        y_ref[...] = y_ref[...] + part


def _ffn(ei, w_in, w_out):
    return pl.pallas_call(
        _ffn_body,
        grid=(E, NF),
        in_specs=[
            pl.BlockSpec((CAP, D), lambda e, f: (e, 0)),
            pl.BlockSpec((1, D, FB), lambda e, f: (e, 0, f)),
            pl.BlockSpec((1, FB, D), lambda e, f: (e, f, 0)),
        ],
        out_specs=pl.BlockSpec((CAP, D), lambda e, f: (e, 0)),
        out_shape=jax.ShapeDtypeStruct((NSLOT, D), jnp.float32),
        compiler_params=pltpu.CompilerParams(
            dimension_semantics=("arbitrary", "arbitrary")),
    )(ei, w_in, w_out)


_CB = 16
_NB = TOK_PER_W // _CB


@functools.partial(
    pl.kernel,
    out_type=jax.ShapeDtypeStruct((T, D), jnp.float32),
    mesh=_SC_MESH,
    scratch_types=[
        pltpu.VMEM((2, 2 * _CB), jnp.int32),
        pltpu.VMEM((2, 2 * _CB, LANES), jnp.float32),
        pltpu.VMEM((2 * _CB, D), jnp.float32),
        pltpu.VMEM((2 * _CB, D), jnp.float32),
        pltpu.VMEM((_CB, D), jnp.float32),
        pltpu.VMEM((_CB, D), jnp.float32),
        pltpu.SemaphoreType.DMA,
        pltpu.SemaphoreType.DMA,
        pltpu.SemaphoreType.DMA,
        pltpu.SemaphoreType.DMA,
    ],
)
def _combine(y_hbm, d12_hbm, w12_hbm, out_hbm,
             didx, wv, rp0, rp1, ob0, ob1, g0, g1, o0, o1):
    cid = lax.axis_index("c")
    sid = lax.axis_index("s")
    wid = sid * NC + cid
    base = wid * TOK_PER_W
    rps = (rp0, rp1)
    obs = (ob0, ob1)
    gsems = (g0, g1)
    osems = (o0, o1)

    def stage(b):
        par = b % 2
        tb = base + b * _CB
        pltpu.sync_copy(d12_hbm.at[pl.ds(2 * tb, 2 * _CB)], didx.at[par])
        pltpu.sync_copy(w12_hbm.at[pl.ds(2 * tb, 2 * _CB)], wv.at[par])
        return pltpu.async_copy(y_hbm.at[didx.at[par]], rps[par], gsems[par])

    cp = stage(0)
    pend = [None, None]
    for b in range(_NB):
        par = b % 2
        cp.wait()
        if b + 1 < _NB:
            cp = stage(b + 1)
        if pend[par] is not None:
            pend[par].wait()
        rp, ob = rps[par], obs[par]
        for i in range(_CB):
            w1row = wv[par, 2 * i, :]
            w2row = wv[par, 2 * i + 1, :]

            def vbody(v, carry, i=i, w1row=w1row, w2row=w2row, rp=rp, ob=ob):
                off = v * (8 * LANES)
                for u in range(8):
                    sl = pl.ds(off + u * LANES, LANES)
                    ob[i, sl] = rp[2 * i, sl] * w1row + rp[2 * i + 1, sl] * w2row
                return carry

            lax.fori_loop(0, D // (8 * LANES), vbody, 0)
        pend[par] = pltpu.async_copy(
            ob, out_hbm.at[pl.ds(base + b * _CB, _CB)], osems[par])
    for h in pend:
        if h is not None:
            h.wait()


def kernel(input_emb, w_router, w_in, w_out):
    x2d = input_emb.reshape(T, D).astype(jnp.float32)
    slot_km, d12, w12 = _routing(x2d, w_router)
    slot_flat = slot_km.reshape(K * T)
    ei = _dispatch(slot_flat, x2d)
    y = _ffn(ei, w_in, w_out)
    out = _combine(y, d12.reshape(K * T), w12.reshape(K * T, LANES))
    return out.reshape(1, T, D)
